# baseline (speedup 1.0000x reference)
import functools

import numpy as np
import jax
import jax.numpy as jnp
from jax import lax
from jax.experimental import pallas as pl
from jax.experimental.pallas import tpu as pltpu
from jax.experimental.pallas import tpu_sc as plsc

_L = 16
_NW = 32
_D = 128


def _np_threefry2x32(k, x0, x1):
    rots = [[13, 15, 26, 6], [17, 29, 16, 24]]
    k0, k1 = np.uint32(k[0]), np.uint32(k[1])
    ks = [k0, k1, np.uint32(k0 ^ k1 ^ np.uint32(0x1BD11BDA))]
    x0 = x0.astype(np.uint32).copy()
    x1 = x1.astype(np.uint32).copy()
    x0 += ks[0]
    x1 += ks[1]

    def rot(x, d):
        return (x << np.uint32(d)) | (x >> np.uint32(32 - d))

    for r in range(5):
        for d in rots[r % 2]:
            x0 += x1
            x1 = rot(x1, d)
            x1 ^= x0
        x0 += ks[(r + 1) % 3]
        x1 += ks[(r + 2) % 3] + np.uint32(r + 1)
    return x0, x1


@functools.lru_cache(maxsize=None)
def _rand_words(bs, n):
    def bits(k, count):
        cnt = np.arange(count, dtype=np.uint64)
        hi = (cnt >> np.uint64(32)).astype(np.uint32)
        lo = (cnt & np.uint64(0xFFFFFFFF)).astype(np.uint32)
        a, b = _np_threefry2x32(k, hi, lo)
        return a ^ b

    his, los = [], []
    for i in range(bs):
        a, b = _np_threefry2x32(
            (0, 42), np.array([0], np.uint32), np.array([i], np.uint32)
        )
        folded = (int(a[0]), int(b[0]))
        a, b = _np_threefry2x32(
            folded, np.array([0, 0], np.uint32), np.array([0, 1], np.uint32)
        )
        k1 = (int(a[0]), int(b[0]))
        k2 = (int(a[1]), int(b[1]))
        his.append(bits(k1, n))
        los.append(bits(k2, n))
    hi = np.concatenate(his).view(np.int32)
    lo = np.concatenate(los).view(np.int32)
    return hi, lo


@functools.partial(jax.jit, static_argnums=(3, 4))
def _sc_embrace(table, mask, hi_lo, seq_len, emb_size):
    bs = mask.shape[0]
    n = bs * emb_size
    epw = n // _NW
    wpb = _NW // bs
    assert epw == _D and emb_size % _D == 0
    mesh = plsc.VectorSubcoreMesh(core_axis_name="c", subcore_axis_name="s")
    mask_chunks = seq_len // _L

    scan_len = seq_len // wpb
    ns = _NW // 2

    @functools.partial(
        pl.kernel,
        out_type=jax.ShapeDtypeStruct((bs, emb_size), jnp.float32),
        mesh=mesh,
        scratch_types=[
            pltpu.VMEM((seq_len,), jnp.int32),
            pltpu.VMEM((2 * epw,), jnp.int32),
            pltpu.VMEM((epw,), jnp.int32),
            pltpu.VMEM((epw, _D), jnp.float32),
            pltpu.VMEM((epw,), jnp.float32),
            pltpu.SemaphoreType.DMA,
        ],
        compiler_params=pltpu.CompilerParams(needs_layout_passes=False),
    )
    def k(table_hbm, mask_hbm, bits_hbm, out_hbm, maskv, bitsv, rowv, rdata,
          outv, sem):
        sid = lax.axis_index("s")
        wid = lax.axis_index("c") * ns + sid
        base = wid * epw
        i_bs = wid // wpb
        j_blk = wid % wpb

        cp_m = pltpu.async_copy(mask_hbm.at[i_bs], maskv, sem)
        cp_h = pltpu.async_copy(
            bits_hbm.at[pl.ds(base, epw)], bitsv.at[pl.ds(0, epw)], sem
        )
        cp_l = pltpu.async_copy(
            bits_hbm.at[pl.ds(n + base, epw)], bitsv.at[pl.ds(epw, epw)], sem
        )
        cp_m.wait()

        lane = lax.iota(jnp.int32, _L)
        acc = jnp.full((_L,), seq_len, jnp.int32)
        for c in range(seq_len // _L):
            m = maskv[pl.ds(c * _L, _L)]
            pos = lane + c * _L
            acc = jnp.minimum(acc, jnp.where(m == 0, pos, seq_len))
        leading = jnp.min(acc, axis=0)
        last_idx = jnp.maximum(leading - 1, 0)
        span = jnp.maximum(last_idx, 1)

        c16 = jnp.int32(65536) % span
        mult = (c16 * c16) % span
        span_v = jnp.full((_L,), span, jnp.int32)
        c16_v = jnp.full((_L,), c16, jnp.int32)
        mult_v = jnp.full((_L,), mult, jnp.int32)
        span_fv = jnp.full((_L,), span, jnp.int32).astype(jnp.float32)
        r0_bits = jnp.full((_L,), 0x7EF311C3, jnp.int32) - plsc.bitcast(
            span_fv, jnp.int32
        )
        recip_v = plsc.bitcast(r0_bits, jnp.float32)
        for _ in range(3):
            recip_v = recip_v * (jnp.float32(2.0) - span_fv * recip_v)
        row_base = jnp.full((_L,), i_bs * seq_len, jnp.int32)

        def redm(x):
            q = (x.astype(jnp.float32) * recip_v).astype(jnp.int32)
            r = x - q * span_v
            r = jnp.where(r < 0, r + span_v, r)
            r = jnp.where(r < 0, r + span_v, r)
            r = jnp.where(r >= span_v, r - span_v, r)
            r = jnp.where(r >= span_v, r - span_v, r)
            return r

        def word_mod(w):
            a = lax.shift_right_logical(w, jnp.int32(16))
            b = w & jnp.int32(0xFFFF)
            return redm(redm(a) * c16_v + redm(b))

        cp_h.wait()
        cp_l.wait()
        for u in range(epw // _L):
            hi = bitsv[pl.ds(u * _L, _L)]
            lo = bitsv[pl.ds(epw + u * _L, _L)]
            idx = redm(word_mod(hi) * mult_v + word_mod(lo))
            rowv[pl.ds(u * _L, _L)] = row_base + idx

        cps = []
        for q in range(epw // _L):
            cps.append(
                pltpu.async_copy(
                    table_hbm.at[
                        rowv.at[pl.ds(q * _L, _L)], pl.ds(j_blk * _D, _D)
                    ],
                    rdata.at[pl.ds(q * _L, _L)],
                    sem,
                )
            )

        for q in range(epw // _L):
            cps[q].wait()
            d = jnp.zeros((_L,), jnp.float32)
            for s in range(_L):
                v = rdata[q * _L + s, pl.ds(q * _L, _L)]
                d = jnp.where(lane == s, v, d)
            outv[pl.ds(q * _L, _L)] = d
        pltpu.sync_copy(outv, out_hbm.at[i_bs, pl.ds(j_blk * epw, epw)])

    return k(table, mask, hi_lo)


def kernel(output_tokens_from_bert, attention_mask):
    bs, seq_len, emb_size = output_tokens_from_bert.shape
    hi, lo = _rand_words(bs, emb_size)
    hi_lo = jnp.asarray(np.concatenate([hi, lo]))
    table = output_tokens_from_bert.reshape(bs * seq_len, emb_size)
    return _sc_embrace(table, attention_mask, hi_lo, seq_len, emb_size)


# speedup vs baseline: 1.0068x; 1.0068x over previous
import functools

import numpy as np
import jax
import jax.numpy as jnp
from jax import lax
from jax.experimental import pallas as pl
from jax.experimental.pallas import tpu as pltpu
from jax.experimental.pallas import tpu_sc as plsc

_L = 16
_NW = 32
_D = 128


def _np_threefry2x32(k, x0, x1):
    rots = [[13, 15, 26, 6], [17, 29, 16, 24]]
    k0, k1 = np.uint32(k[0]), np.uint32(k[1])
    ks = [k0, k1, np.uint32(k0 ^ k1 ^ np.uint32(0x1BD11BDA))]
    x0 = x0.astype(np.uint32).copy()
    x1 = x1.astype(np.uint32).copy()
    x0 += ks[0]
    x1 += ks[1]

    def rot(x, d):
        return (x << np.uint32(d)) | (x >> np.uint32(32 - d))

    for r in range(5):
        for d in rots[r % 2]:
            x0 += x1
            x1 = rot(x1, d)
            x1 ^= x0
        x0 += ks[(r + 1) % 3]
        x1 += ks[(r + 2) % 3] + np.uint32(r + 1)
    return x0, x1


@functools.lru_cache(maxsize=None)
def _rand_words(bs, n):
    def bits(k, count):
        cnt = np.arange(count, dtype=np.uint64)
        hi = (cnt >> np.uint64(32)).astype(np.uint32)
        lo = (cnt & np.uint64(0xFFFFFFFF)).astype(np.uint32)
        a, b = _np_threefry2x32(k, hi, lo)
        return a ^ b

    his, los = [], []
    for i in range(bs):
        a, b = _np_threefry2x32(
            (0, 42), np.array([0], np.uint32), np.array([i], np.uint32)
        )
        folded = (int(a[0]), int(b[0]))
        a, b = _np_threefry2x32(
            folded, np.array([0, 0], np.uint32), np.array([0, 1], np.uint32)
        )
        k1 = (int(a[0]), int(b[0]))
        k2 = (int(a[1]), int(b[1]))
        his.append(bits(k1, n))
        los.append(bits(k2, n))
    hi = np.concatenate(his).view(np.int32)
    lo = np.concatenate(los).view(np.int32)
    return hi, lo


@functools.partial(jax.jit, static_argnums=(3, 4))
def _sc_embrace(table, mask, hi_lo, seq_len, emb_size):
    bs = mask.shape[0]
    n = bs * emb_size
    epw = n // _NW
    wpb = _NW // bs
    assert epw == _D and emb_size % _D == 0
    mesh = plsc.VectorSubcoreMesh(core_axis_name="c", subcore_axis_name="s")
    mask_chunks = seq_len // _L

    scan_len = seq_len // wpb
    ns = _NW // 2

    @functools.partial(
        pl.kernel,
        out_type=jax.ShapeDtypeStruct((bs, emb_size), jnp.float32),
        mesh=mesh,
        scratch_types=[
            pltpu.VMEM((seq_len,), jnp.int32),
            pltpu.VMEM((2 * epw,), jnp.int32),
            pltpu.VMEM((epw,), jnp.int32),
            pltpu.VMEM((epw, _D), jnp.float32),
            pltpu.VMEM((epw,), jnp.float32),
            pltpu.SemaphoreType.DMA,
        ],
        compiler_params=pltpu.CompilerParams(needs_layout_passes=False),
    )
    def k(table_hbm, mask_hbm, bits_hbm, out_hbm, maskv, bitsv, rowv, rdata,
          outv, sem):
        sid = lax.axis_index("s")
        wid = lax.axis_index("c") * ns + sid
        base = wid * epw
        i_bs = wid // wpb
        j_blk = wid % wpb

        cp_m = pltpu.async_copy(mask_hbm.at[i_bs], maskv, sem)
        cp_h = pltpu.async_copy(
            bits_hbm.at[pl.ds(base, epw)], bitsv.at[pl.ds(0, epw)], sem
        )
        cp_l = pltpu.async_copy(
            bits_hbm.at[pl.ds(n + base, epw)], bitsv.at[pl.ds(epw, epw)], sem
        )
        cp_m.wait()

        lane = lax.iota(jnp.int32, _L)
        acc = jnp.full((_L,), seq_len, jnp.int32)
        for c in range(seq_len // _L):
            m = maskv[pl.ds(c * _L, _L)]
            pos = lane + c * _L
            acc = jnp.minimum(acc, jnp.where(m == 0, pos, seq_len))
        leading = jnp.min(acc, axis=0)
        last_idx = jnp.maximum(leading - 1, 0)
        span = jnp.maximum(last_idx, 1)

        c16 = jnp.int32(65536) % span
        mult = (c16 * c16) % span
        span_v = jnp.full((_L,), span, jnp.int32)
        c16_v = jnp.full((_L,), c16, jnp.int32)
        mult_v = jnp.full((_L,), mult, jnp.int32)
        span_fv = jnp.full((_L,), span, jnp.int32).astype(jnp.float32)
        r0_bits = jnp.full((_L,), 0x7EF311C3, jnp.int32) - plsc.bitcast(
            span_fv, jnp.int32
        )
        recip_v = plsc.bitcast(r0_bits, jnp.float32)
        for _ in range(3):
            recip_v = recip_v * (jnp.float32(2.0) - span_fv * recip_v)
        row_base = jnp.full((_L,), i_bs * seq_len, jnp.int32)

        def redm(x):
            q = (x.astype(jnp.float32) * recip_v).astype(jnp.int32)
            r = x - q * span_v
            r = jnp.where(r < 0, r + span_v, r)
            r = jnp.where(r < 0, r + span_v, r)
            r = jnp.where(r >= span_v, r - span_v, r)
            r = jnp.where(r >= span_v, r - span_v, r)
            return r

        def word_mod(w):
            a = lax.shift_right_logical(w, jnp.int32(16))
            b = w & jnp.int32(0xFFFF)
            return redm(redm(a) * c16_v + redm(b))

        cp_h.wait()
        cp_l.wait()
        for u in range(epw // _L):
            hi = bitsv[pl.ds(u * _L, _L)]
            lo = bitsv[pl.ds(epw + u * _L, _L)]
            idx = redm(word_mod(hi) * mult_v + word_mod(lo))
            rowv[pl.ds(u * _L, _L)] = row_base + idx

        pltpu.async_copy(
            table_hbm.at[rowv, pl.ds(j_blk * _D, _D)], rdata, sem
        ).wait()

        for q in range(epw // _L):
            d = jnp.zeros((_L,), jnp.float32)
            for s in range(_L):
                v = rdata[q * _L + s, pl.ds(q * _L, _L)]
                d = jnp.where(lane == s, v, d)
            outv[pl.ds(q * _L, _L)] = d
        pltpu.sync_copy(outv, out_hbm.at[i_bs, pl.ds(j_blk * epw, epw)])

    return k(table, mask, hi_lo)


def kernel(output_tokens_from_bert, attention_mask):
    bs, seq_len, emb_size = output_tokens_from_bert.shape
    hi, lo = _rand_words(bs, emb_size)
    hi_lo = jnp.asarray(np.concatenate([hi, lo]))
    table = output_tokens_from_bert.reshape(bs * seq_len, emb_size)
    return _sc_embrace(table, attention_mask, hi_lo, seq_len, emb_size)
